# baseline (speedup 1.0000x reference)
import functools

import jax
import jax.numpy as jnp
from jax.experimental import pallas as pl
from jax.experimental.pallas import tpu as pltpu
from jax.experimental.pallas import tpu_sc as plsc

_B, _N, _K, _S, _DS = 16, 1024, 32, 512, 512


def _mm(a, b):
    return jax.lax.dot_general(a, b, (((a.ndim - 1,), (0,)), ((), ())),
                               preferred_element_type=jnp.float32)


def _mm_tn(a, b):
    return jax.lax.dot_general(a, b, (((0,), (0,)), ((), ())),
                               preferred_element_type=jnp.float32)


def _mm_nt(a, b):
    return jax.lax.dot_general(a, b, (((1,), (1,)), ((), ())),
                               preferred_element_type=jnp.float32)


def _topk_min_idx(dist, k, base):
    m, n = dist.shape
    cols = jax.lax.broadcasted_iota(jnp.int32, (m, n), 1)
    keys = jnp.bitwise_or(
        jnp.bitwise_and(jax.lax.bitcast_convert_type(dist, jnp.int32),
                        jnp.int32(-1024)), cols)
    kio = jax.lax.broadcasted_iota(jnp.int32, (m, k), 1)
    idx0 = jnp.zeros((m, k), jnp.int32)
    big = jnp.int32(0x7fffffff)

    def body(j, carry):
        kk, idx = carry
        mn = jnp.min(kk, axis=1, keepdims=True)
        kk = jnp.where(kk == mn, big, kk)
        idx = jnp.where(kio == j, jnp.bitwise_and(mn, 1023) + base, idx)
        return kk, idx

    _, idx = jax.lax.fori_loop(0, k, body, (keys, idx0))
    return idx


def _k1_body(xyz_ref, wsum_ref, wq_ref, b1_ref, idx_ref, a_ref, qb_ref):
    b = pl.program_id(0)
    x = xyz_ref[0]
    sq = jnp.sum(x * x, axis=1, keepdims=True)
    dist = sq - 2.0 * _mm_nt(x, x) + jnp.transpose(sq)
    idx_ref[0] = _topk_min_idx(dist, _K, b * _N)
    a = _mm(x, wsum_ref[...])
    a_ref[0] = jnp.concatenate([a, jnp.zeros_like(a)], axis=1)
    qb_ref[0] = _mm(x, wq_ref[...]) - b1_ref[...]


def _oa_block(x, wq, bq, wk, bk, wv, bv, wt, bt):
    q = _mm(x, wq) + bq
    k = _mm(x, wk) + bk
    v = _mm(x, wv) + bv
    e = _mm_nt(q, k)
    a = jax.nn.softmax(e, axis=-1)
    a = a / (1e-9 + jnp.sum(a, axis=0, keepdims=True))
    xr = _mm_tn(a, v)
    y = jnp.maximum(_mm(x - xr, wt) + bt, 0.0)
    return x + y


def _k2_body(g1_ref, qb_ref, w2_ref, b2_ref, *rest):
    (oa_refs, fea_ref) = rest[:-1], rest[-1]
    g = g1_ref[...][:, :64]
    qb = qb_ref[0]
    h1 = jnp.maximum(g.reshape(_N, _K, 64) - qb[:, None, :], 0.0)
    h2 = jnp.maximum(_mm(h1.reshape(_N * _K, 64), w2_ref[...]) + b2_ref[...],
                     0.0)
    pts = jnp.max(h2.reshape(_N, _K, 64), axis=1)
    f = pts
    outs = []
    for i in range(3):
        ws = [r[...] for r in oa_refs[8 * i:8 * (i + 1)]]
        f = _oa_block(f, *ws)
        outs.append(f)
    fea_ref[0] = jnp.concatenate(outs, axis=1)


def _k3_body(fea_ref, xyz_ref, ws_ref, bs_ref, wux_ref, wuf_ref, wb3_ref,
             wd_ref, bm1_ref, idx_ref, u_ref, v_ref, nx_ref, cos_ref):
    b = pl.program_id(0)
    fea = fea_ref[0]
    x = xyz_ref[0]
    z = _mm(fea, ws_ref[...]) + bs_ref[...]
    ez = jnp.exp(z - jnp.max(z, axis=0, keepdims=True))
    pt = ez / jnp.sum(ez, axis=0, keepdims=True)
    nxyz = _mm_tn(pt, x)
    cfea = _mm_tn(pt, fea)
    rn2 = jnp.sum(pt * pt, axis=0, keepdims=True)
    inv = 1.0 / (jnp.sqrt(rn2) + 1e-9)
    vs = _mm_nt(pt, inv)
    cpart = (jnp.sum(vs * vs) - jnp.sum(rn2 * inv * inv)) / (
        _B * _S * (_S - 1))

    @pl.when(b == 0)
    def _():
        cos_ref[...] = jnp.zeros_like(cos_ref)

    cos_ref[...] = cos_ref[...] + jnp.broadcast_to(cpart, (1, 1))

    sqq = jnp.sum(nxyz * nxyz, axis=1, keepdims=True)
    sqp = jnp.sum(x * x, axis=1, keepdims=True)
    dist = sqq - 2.0 * _mm_nt(nxyz, x) + jnp.transpose(sqp)
    idx_ref[0] = _topk_min_idx(dist, _K, b * _N)
    uu = _mm(x, wux_ref[...]) + _mm(fea, wuf_ref[...])
    ui = jax.lax.bitcast_convert_type(uu, jnp.int32) + jnp.int32(0x8000)
    u_ref[0] = jnp.bitwise_or(
        jax.lax.shift_right_logical(ui[:, :128], 16),
        jnp.bitwise_and(ui[:, 128:], jnp.int32(-65536)))
    v_ref[0] = _mm(cfea, wd_ref[...]) - _mm(nxyz, wb3_ref[...]) + bm1_ref[...]
    nx_ref[0] = nxyz


def _k4_body(g2_ref, v_ref, wm2_ref, bm2_ref, out_ref):
    rows = v_ref.shape[1]
    gp = g2_ref[...]
    g = jnp.concatenate([
        jax.lax.bitcast_convert_type(jnp.left_shift(gp, 16), jnp.float32),
        jax.lax.bitcast_convert_type(
            jnp.bitwise_and(gp, jnp.int32(-65536)), jnp.float32),
    ], axis=1)
    v = v_ref[0]
    h1 = jnp.maximum(g.reshape(rows, _K, 256) + v[:, None, :], 0.0)
    h2 = jnp.maximum(
        _mm(h1.reshape(rows * _K, 256), wm2_ref[...]) + bm2_ref[...], 0.0)
    out_ref[0] = jnp.max(h2.reshape(rows, _K, 256), axis=1)


def _k5_body(nx_ref, wc1, bc1, wc2, bc2, wc3, bc3, wf1, bf1, wf2, bf2, wf3,
             bf3, out_ref):
    x = nx_ref[0]
    h = jnp.maximum(_mm(x, wc1[...]) + bc1[...], 0.0)
    h = jnp.maximum(_mm(h, wc2[...]) + bc2[...], 0.0)
    h = jnp.maximum(_mm(h, wc3[...]) + bc3[...], 0.0)
    g = jnp.max(h, axis=0, keepdims=True)
    f = jnp.maximum(_mm(g, wf1[...]) + bf1[...], 0.0)
    f = jnp.maximum(_mm(f, wf2[...]) + bf2[...], 0.0)
    o = _mm(f, wf3[...]) + bf3[...]
    o = o - jnp.max(o, axis=1, keepdims=True)
    out_ref[0] = o - jnp.log(jnp.sum(jnp.exp(o), axis=1, keepdims=True))


def _gather_rows(table, idx, window):
    n_idx = idx.shape[0]
    width = table.shape[1]
    idx2d = idx.reshape(1, n_idx)
    mesh = plsc.VectorSubcoreMesh(core_axis_name="core",
                                  subcore_axis_name="subcore")

    @functools.partial(
        pl.kernel,
        out_type=jax.ShapeDtypeStruct((n_idx, width), table.dtype),
        mesh=mesh)
    def _g(x_hbm, i_hbm, o_hbm):
        def body(i_vmem, o_vmem):
            pltpu.sync_copy(x_hbm.at[i_vmem.at[0]], o_vmem)

        pltpu.emit_pipeline(
            body,
            grid=(n_idx // window,),
            in_specs=[pl.BlockSpec((1, window), index_map=lambda i: (0, i))],
            out_specs=[
                pl.BlockSpec((window, width), index_map=lambda i: (i, 0))
            ],
            core_axis_name=("core", "subcore"),
            dimension_semantics=(pltpu.PARALLEL,),
        )(i_hbm, o_hbm)

    return _g(table, idx2d)


def _full(shape):
    return pl.BlockSpec(shape, lambda b: (0,) * len(shape))


def _batched(shape):
    return pl.BlockSpec((1,) + shape, lambda b: (b,) + (0,) * len(shape))


_CHUNKS = 2


def kernel(xyz, params):
    p = params
    xyz_t = jnp.transpose(xyz, (0, 2, 1)).astype(jnp.float32)
    f32 = jnp.float32
    nb = _B // _CHUNKS

    def row(v):
        return v.reshape(1, -1).astype(f32)

    w1a = p['W_re1'][:3].astype(f32)
    wsum = (p['W_re1'][:3] + p['W_re1'][3:]).astype(f32)
    b_re1 = row(p['b_re1'])
    w_re2 = p['W_re2'].astype(f32)
    b_re2 = row(p['b_re2'])
    ws = p['Ws'].astype(f32)
    bs = row(p['bs'])
    wm1 = p['Wm1'].astype(f32)
    wux, wb3 = (wm1[:3] + wm1[3:6]), wm1[3:6]
    wuf, wd = wm1[6:198], wm1[198:390]
    bm1 = row(p['bm1'])
    wm2 = p['Wm2'].astype(f32)
    bm2 = row(p['bm2'])

    oa_params = []
    oa_specs = []
    for i in (1, 2, 3):
        for nm, sh in (('Wq', (64, 16)), ('bq', (1, 16)), ('Wk', (64, 16)),
                       ('bk', (1, 16)), ('Wv', (64, 64)), ('bv', (1, 64)),
                       ('Wt', (64, 64)), ('bt', (1, 64))):
            key = 'oa%d_%s' % (i, nm)
            arr = p[key]
            if nm.startswith('b'):
                arr = row(arr)
            oa_params.append(arr.astype(f32))
            oa_specs.append(_full(sh))

    k1 = pl.pallas_call(
        _k1_body,
        grid=(nb,),
        in_specs=[_batched((_N, 3)), _full((3, 64)), _full((3, 64)),
                  _full((1, 64))],
        out_specs=[_batched((_N, _K)), _batched((_N, 128)), _batched((_N, 64))],
        out_shape=[
            jax.ShapeDtypeStruct((nb, _N, _K), jnp.int32),
            jax.ShapeDtypeStruct((nb, _N, 128), f32),
            jax.ShapeDtypeStruct((nb, _N, 64), f32),
        ],
    )
    k2 = pl.pallas_call(
        _k2_body,
        grid=(nb,),
        in_specs=[pl.BlockSpec((_N * _K, 128), lambda b: (b, 0)),
                  _batched((_N, 64)), _full((64, 64)), _full((1, 64))]
        + oa_specs,
        out_specs=_batched((_N, 192)),
        out_shape=jax.ShapeDtypeStruct((nb, _N, 192), f32),
    )
    k3 = pl.pallas_call(
        _k3_body,
        grid=(nb,),
        in_specs=[_batched((_N, 192)), _batched((_N, 3)), _full((192, _DS)),
                  _full((1, _DS)), _full((3, 256)), _full((192, 256)),
                  _full((3, 256)), _full((192, 256)), _full((1, 256))],
        out_specs=[_batched((_S, _K)), _batched((_N, 128)),
                   _batched((_S, 256)), _batched((_S, 3)),
                   pl.BlockSpec((1, 1), lambda b: (0, 0))],
        out_shape=[
            jax.ShapeDtypeStruct((nb, _S, _K), jnp.int32),
            jax.ShapeDtypeStruct((nb, _N, 128), jnp.int32),
            jax.ShapeDtypeStruct((nb, _S, 256), f32),
            jax.ShapeDtypeStruct((nb, _S, 3), f32),
            jax.ShapeDtypeStruct((1, 1), f32),
        ],
    )
    st = 8
    rows = _S // st
    k4 = pl.pallas_call(
        _k4_body,
        grid=(nb, st),
        in_specs=[
            pl.BlockSpec((rows * _K, 128), lambda b, t: (b * st + t, 0)),
            pl.BlockSpec((1, rows, 256), lambda b, t: (b, t, 0)),
            pl.BlockSpec((256, 256), lambda b, t: (0, 0)),
            pl.BlockSpec((1, 256), lambda b, t: (0, 0)),
        ],
        out_specs=pl.BlockSpec((1, rows, 256), lambda b, t: (b, t, 0)),
        out_shape=jax.ShapeDtypeStruct((nb, _S, 256), f32),
    )

    def run_chunk(xc):
        idx1, a_t, qb = k1(xc, wsum, w1a, b_re1)
        g1 = _gather_rows(a_t.reshape(nb * _N, 128), idx1.reshape(-1), 256)
        fea = k2(g1, qb, w_re2, b_re2, *oa_params)
        idx2, u, v, nxyz_c, cos_c = k3(fea, xc, ws, bs, wux, wuf, wb3, wd,
                                       bm1)
        g2 = _gather_rows(u.reshape(nb * _N, 128), idx2.reshape(-1), 256)
        npts_c = k4(g2, v, wm2, bm2)
        return nxyz_c, npts_c, cos_c

    outs = [run_chunk(xyz_t[c * nb:(c + 1) * nb]) for c in range(_CHUNKS)]
    nxyz = jnp.concatenate([o[0] for o in outs], axis=0)
    npts = jnp.concatenate([o[1] for o in outs], axis=0)
    cosac = outs[0][2]
    for o in outs[1:]:
        cosac = cosac + o[2]

    k5 = pl.pallas_call(
        _k5_body,
        grid=(_B,),
        in_specs=[_batched((_S, 3)), _full((3, 64)), _full((1, 64)),
                  _full((64, 128)), _full((1, 128)), _full((128, 1024)),
                  _full((1, 1024)), _full((1024, 512)), _full((1, 512)),
                  _full((512, 256)), _full((1, 256)), _full((256, 40)),
                  _full((1, 40))],
        out_specs=_batched((1, 40)),
        out_shape=jax.ShapeDtypeStruct((_B, 1, 40), f32),
    )
    cls = k5(nxyz, p['Wc1'].astype(f32), row(p['bc1']), p['Wc2'].astype(f32),
             row(p['bc2']), p['Wc3'].astype(f32), row(p['bc3']),
             p['Wf1'].astype(f32), row(p['bf1']), p['Wf2'].astype(f32),
             row(p['bf2']), p['Wf3'].astype(f32), row(p['bf3']))

    l1_xyz = jnp.transpose(nxyz, (0, 2, 1))
    l1_points = jnp.transpose(npts, (0, 2, 1))
    return l1_xyz, cls.reshape(_B, 40), cosac[0, 0], l1_points


# speedup vs baseline: 1.0136x; 1.0136x over previous
import functools

import jax
import jax.numpy as jnp
from jax.experimental import pallas as pl
from jax.experimental.pallas import tpu as pltpu
from jax.experimental.pallas import tpu_sc as plsc

_B, _N, _K, _S, _DS = 16, 1024, 32, 512, 512


def _mm(a, b):
    return jax.lax.dot_general(a, b, (((a.ndim - 1,), (0,)), ((), ())),
                               preferred_element_type=jnp.float32)


def _mm_tn(a, b):
    return jax.lax.dot_general(a, b, (((0,), (0,)), ((), ())),
                               preferred_element_type=jnp.float32)


def _mm_nt(a, b):
    return jax.lax.dot_general(a, b, (((1,), (1,)), ((), ())),
                               preferred_element_type=jnp.float32)


def _topk_min_idx(dist, k, base):
    m, n = dist.shape
    cols = jax.lax.broadcasted_iota(jnp.int32, (m, n), 1)
    keys = jnp.bitwise_or(
        jnp.bitwise_and(jax.lax.bitcast_convert_type(dist, jnp.int32),
                        jnp.int32(-1024)), cols)
    kio = jax.lax.broadcasted_iota(jnp.int32, (m, k), 1)
    idx0 = jnp.zeros((m, k), jnp.int32)
    big = jnp.int32(0x7fffffff)

    def body(j, carry):
        kk, idx = carry
        mn = jnp.min(kk, axis=1, keepdims=True)
        kk = jnp.where(kk == mn, big, kk)
        idx = jnp.where(kio == j, jnp.bitwise_and(mn, 1023) + base, idx)
        return kk, idx

    _, idx = jax.lax.fori_loop(0, k, body, (keys, idx0))
    return idx


def _k1_body(xyz_ref, wsum_ref, wq_ref, b1_ref, idx_ref, a_ref, qb_ref):
    b = pl.program_id(0)
    x = xyz_ref[0]
    sq = jnp.sum(x * x, axis=1, keepdims=True)
    dist = sq - 2.0 * _mm_nt(x, x) + jnp.transpose(sq)
    idx_ref[0] = _topk_min_idx(dist, _K, b * _N)
    a = _mm(x, wsum_ref[...])
    a_ref[0] = jnp.concatenate([a, jnp.zeros_like(a)], axis=1)
    qb_ref[0] = _mm(x, wq_ref[...]) - b1_ref[...]


def _oa_block(x, wq, bq, wk, bk, wv, bv, wt, bt):
    q = _mm(x, wq) + bq
    k = _mm(x, wk) + bk
    v = _mm(x, wv) + bv
    e = _mm_nt(q, k)
    a = jax.nn.softmax(e, axis=-1)
    a = a / (1e-9 + jnp.sum(a, axis=0, keepdims=True))
    xr = _mm_tn(a, v)
    y = jnp.maximum(_mm(x - xr, wt) + bt, 0.0)
    return x + y


def _k2_body(g1_ref, qb_ref, w2_ref, b2_ref, *rest):
    (oa_refs, fea_ref) = rest[:-1], rest[-1]
    g = g1_ref[...][:, :64]
    qb = qb_ref[0]
    h1 = jnp.maximum(g.reshape(_N, _K, 64) - qb[:, None, :], 0.0)
    h2 = jnp.maximum(_mm(h1.reshape(_N * _K, 64), w2_ref[...]) + b2_ref[...],
                     0.0)
    pts = jnp.max(h2.reshape(_N, _K, 64), axis=1)
    f = pts
    outs = []
    for i in range(3):
        ws = [r[...] for r in oa_refs[8 * i:8 * (i + 1)]]
        f = _oa_block(f, *ws)
        outs.append(f)
    fea_ref[0] = jnp.concatenate(outs, axis=1)


def _k3_body(fea_ref, xyz_ref, ws_ref, bs_ref, wux_ref, wuf_ref, wb3_ref,
             wd_ref, bm1_ref, idx_ref, u_ref, v_ref, nx_ref, cos_ref):
    b = pl.program_id(0)
    fea = fea_ref[0]
    x = xyz_ref[0]
    z = _mm(fea, ws_ref[...]) + bs_ref[...]
    ez = jnp.exp(z - jnp.max(z, axis=0, keepdims=True))
    pt = ez / jnp.sum(ez, axis=0, keepdims=True)
    nxyz = _mm_tn(pt, x)
    cfea = _mm_tn(pt, fea)
    rn2 = jnp.sum(pt * pt, axis=0, keepdims=True)
    inv = 1.0 / (jnp.sqrt(rn2) + 1e-9)
    vs = _mm_nt(pt, inv)
    cpart = (jnp.sum(vs * vs) - jnp.sum(rn2 * inv * inv)) / (
        _B * _S * (_S - 1))

    @pl.when(b == 0)
    def _():
        cos_ref[...] = jnp.zeros_like(cos_ref)

    cos_ref[...] = cos_ref[...] + jnp.broadcast_to(cpart, (1, 1))

    sqq = jnp.sum(nxyz * nxyz, axis=1, keepdims=True)
    sqp = jnp.sum(x * x, axis=1, keepdims=True)
    dist = sqq - 2.0 * _mm_nt(nxyz, x) + jnp.transpose(sqp)
    idx_ref[0] = _topk_min_idx(dist, _K, b * _N)
    uu = _mm(x, wux_ref[...]) + _mm(fea, wuf_ref[...])
    ui = jax.lax.bitcast_convert_type(uu, jnp.int32) + jnp.int32(0x8000)
    u_ref[0] = jnp.bitwise_or(
        jax.lax.shift_right_logical(ui[:, :128], 16),
        jnp.bitwise_and(ui[:, 128:], jnp.int32(-65536)))
    v_ref[0] = _mm(cfea, wd_ref[...]) - _mm(nxyz, wb3_ref[...]) + bm1_ref[...]
    nx_ref[0] = nxyz


def _k4_body(g2_ref, v_ref, wm2_ref, bm2_ref, out_ref):
    rows = v_ref.shape[1]
    gp = g2_ref[...]
    g = jnp.concatenate([
        jax.lax.bitcast_convert_type(jnp.left_shift(gp, 16), jnp.float32),
        jax.lax.bitcast_convert_type(
            jnp.bitwise_and(gp, jnp.int32(-65536)), jnp.float32),
    ], axis=1)
    v = v_ref[0]
    h1 = jnp.maximum(g.reshape(rows, _K, 256) + v[:, None, :], 0.0)
    h2 = jnp.maximum(
        _mm(h1.reshape(rows * _K, 256), wm2_ref[...]) + bm2_ref[...], 0.0)
    out_ref[0] = jnp.max(h2.reshape(rows, _K, 256), axis=1)


def _k5_body(nx_ref, wc1, bc1, wc2, bc2, wc3, bc3, wf1, bf1, wf2, bf2, wf3,
             bf3, out_ref):
    x = nx_ref[0]
    h = jnp.maximum(_mm(x, wc1[...]) + bc1[...], 0.0)
    h = jnp.maximum(_mm(h, wc2[...]) + bc2[...], 0.0)
    h = jnp.maximum(_mm(h, wc3[...]) + bc3[...], 0.0)
    g = jnp.max(h, axis=0, keepdims=True)
    f = jnp.maximum(_mm(g, wf1[...]) + bf1[...], 0.0)
    f = jnp.maximum(_mm(f, wf2[...]) + bf2[...], 0.0)
    o = _mm(f, wf3[...]) + bf3[...]
    o = o - jnp.max(o, axis=1, keepdims=True)
    out_ref[0] = o - jnp.log(jnp.sum(jnp.exp(o), axis=1, keepdims=True))


def _gather_rows(table, idx, window):
    n_idx = idx.shape[0]
    width = table.shape[1]
    idx2d = idx.reshape(1, n_idx)
    mesh = plsc.VectorSubcoreMesh(core_axis_name="core",
                                  subcore_axis_name="subcore")

    @functools.partial(
        pl.kernel,
        out_type=jax.ShapeDtypeStruct((n_idx, width), table.dtype),
        mesh=mesh)
    def _g(x_hbm, i_hbm, o_hbm):
        def body(i_vmem, o_vmem):
            pltpu.sync_copy(x_hbm.at[i_vmem.at[0]], o_vmem)

        pltpu.emit_pipeline(
            body,
            grid=(n_idx // window,),
            in_specs=[pl.BlockSpec((1, window), index_map=lambda i: (0, i))],
            out_specs=[
                pl.BlockSpec((window, width), index_map=lambda i: (i, 0))
            ],
            core_axis_name=("core", "subcore"),
            dimension_semantics=(pltpu.PARALLEL,),
        )(i_hbm, o_hbm)

    return _g(table, idx2d)


def _full(shape):
    return pl.BlockSpec(shape, lambda b: (0,) * len(shape))


def _batched(shape):
    return pl.BlockSpec((1,) + shape, lambda b: (b,) + (0,) * len(shape))


_CHUNKS = 2


def kernel(xyz, params):
    p = params
    xyz_t = jnp.transpose(xyz, (0, 2, 1)).astype(jnp.float32)
    f32 = jnp.float32
    nb = _B // _CHUNKS

    def row(v):
        return v.reshape(1, -1).astype(f32)

    w1a = p['W_re1'][:3].astype(f32)
    wsum = (p['W_re1'][:3] + p['W_re1'][3:]).astype(f32)
    b_re1 = row(p['b_re1'])
    w_re2 = p['W_re2'].astype(f32)
    b_re2 = row(p['b_re2'])
    ws = p['Ws'].astype(f32)
    bs = row(p['bs'])
    wm1 = p['Wm1'].astype(f32)
    wux, wb3 = (wm1[:3] + wm1[3:6]), wm1[3:6]
    wuf, wd = wm1[6:198], wm1[198:390]
    bm1 = row(p['bm1'])
    wm2 = p['Wm2'].astype(f32)
    bm2 = row(p['bm2'])

    oa_params = []
    oa_specs = []
    for i in (1, 2, 3):
        for nm, sh in (('Wq', (64, 16)), ('bq', (1, 16)), ('Wk', (64, 16)),
                       ('bk', (1, 16)), ('Wv', (64, 64)), ('bv', (1, 64)),
                       ('Wt', (64, 64)), ('bt', (1, 64))):
            key = 'oa%d_%s' % (i, nm)
            arr = p[key]
            if nm.startswith('b'):
                arr = row(arr)
            oa_params.append(arr.astype(f32))
            oa_specs.append(_full(sh))

    k1 = pl.pallas_call(
        _k1_body,
        grid=(nb,),
        in_specs=[_batched((_N, 3)), _full((3, 64)), _full((3, 64)),
                  _full((1, 64))],
        out_specs=[_batched((_N, _K)), _batched((_N, 128)), _batched((_N, 64))],
        out_shape=[
            jax.ShapeDtypeStruct((nb, _N, _K), jnp.int32),
            jax.ShapeDtypeStruct((nb, _N, 128), f32),
            jax.ShapeDtypeStruct((nb, _N, 64), f32),
        ],
    )
    k2 = pl.pallas_call(
        _k2_body,
        grid=(nb,),
        in_specs=[pl.BlockSpec((_N * _K, 128), lambda b: (b, 0)),
                  _batched((_N, 64)), _full((64, 64)), _full((1, 64))]
        + oa_specs,
        out_specs=_batched((_N, 192)),
        out_shape=jax.ShapeDtypeStruct((nb, _N, 192), f32),
    )
    k3 = pl.pallas_call(
        _k3_body,
        grid=(nb,),
        in_specs=[_batched((_N, 192)), _batched((_N, 3)), _full((192, _DS)),
                  _full((1, _DS)), _full((3, 256)), _full((192, 256)),
                  _full((3, 256)), _full((192, 256)), _full((1, 256))],
        out_specs=[_batched((_S, _K)), _batched((_N, 128)),
                   _batched((_S, 256)), _batched((_S, 3)),
                   pl.BlockSpec((1, 1), lambda b: (0, 0))],
        out_shape=[
            jax.ShapeDtypeStruct((nb, _S, _K), jnp.int32),
            jax.ShapeDtypeStruct((nb, _N, 128), jnp.int32),
            jax.ShapeDtypeStruct((nb, _S, 256), f32),
            jax.ShapeDtypeStruct((nb, _S, 3), f32),
            jax.ShapeDtypeStruct((1, 1), f32),
        ],
    )
    st = 2
    rows = _S // st
    k4 = pl.pallas_call(
        _k4_body,
        grid=(nb, st),
        in_specs=[
            pl.BlockSpec((rows * _K, 128), lambda b, t: (b * st + t, 0)),
            pl.BlockSpec((1, rows, 256), lambda b, t: (b, t, 0)),
            pl.BlockSpec((256, 256), lambda b, t: (0, 0)),
            pl.BlockSpec((1, 256), lambda b, t: (0, 0)),
        ],
        out_specs=pl.BlockSpec((1, rows, 256), lambda b, t: (b, t, 0)),
        out_shape=jax.ShapeDtypeStruct((nb, _S, 256), f32),
    )

    def run_chunk(xc):
        idx1, a_t, qb = k1(xc, wsum, w1a, b_re1)
        g1 = _gather_rows(a_t.reshape(nb * _N, 128), idx1.reshape(-1), 256)
        fea = k2(g1, qb, w_re2, b_re2, *oa_params)
        idx2, u, v, nxyz_c, cos_c = k3(fea, xc, ws, bs, wux, wuf, wb3, wd,
                                       bm1)
        g2 = _gather_rows(u.reshape(nb * _N, 128), idx2.reshape(-1), 256)
        npts_c = k4(g2, v, wm2, bm2)
        return nxyz_c, npts_c, cos_c

    outs = [run_chunk(xyz_t[c * nb:(c + 1) * nb]) for c in range(_CHUNKS)]
    nxyz = jnp.concatenate([o[0] for o in outs], axis=0)
    npts = jnp.concatenate([o[1] for o in outs], axis=0)
    cosac = outs[0][2]
    for o in outs[1:]:
        cosac = cosac + o[2]

    k5 = pl.pallas_call(
        _k5_body,
        grid=(_B,),
        in_specs=[_batched((_S, 3)), _full((3, 64)), _full((1, 64)),
                  _full((64, 128)), _full((1, 128)), _full((128, 1024)),
                  _full((1, 1024)), _full((1024, 512)), _full((1, 512)),
                  _full((512, 256)), _full((1, 256)), _full((256, 40)),
                  _full((1, 40))],
        out_specs=_batched((1, 40)),
        out_shape=jax.ShapeDtypeStruct((_B, 1, 40), f32),
    )
    cls = k5(nxyz, p['Wc1'].astype(f32), row(p['bc1']), p['Wc2'].astype(f32),
             row(p['bc2']), p['Wc3'].astype(f32), row(p['bc3']),
             p['Wf1'].astype(f32), row(p['bf1']), p['Wf2'].astype(f32),
             row(p['bf2']), p['Wf3'].astype(f32), row(p['bf3']))

    l1_xyz = jnp.transpose(nxyz, (0, 2, 1))
    l1_points = jnp.transpose(npts, (0, 2, 1))
    return l1_xyz, cls.reshape(_B, 40), cosac[0, 0], l1_points
